# baseline (speedup 1.0000x reference)
import functools

import jax
import jax.numpy as jnp
from jax import lax
from jax.experimental import pallas as pl
from jax.experimental.pallas import tpu as pltpu
from jax.experimental.pallas import tpu_sc as plsc

NSC = 2
NTILE = 16
NW = NSC * NTILE
CH = 128
F32 = jnp.float32


def _sc_mesh():
    return plsc.VectorSubcoreMesh(core_axis_name="c", subcore_axis_name="s")


def _degree_call(dstp, npad, k_chunks):
    rpt = npad // NTILE

    @functools.partial(
        pl.kernel,
        mesh=_sc_mesh(),
        out_type=jax.ShapeDtypeStruct((NSC, npad), F32),
        scratch_types=[
            pltpu.VMEM((k_chunks, CH), jnp.int32),
            pltpu.VMEM((rpt,), F32),
            pltpu.VMEM((CH,), F32),
            pltpu.VMEM_SHARED((npad,), F32),
        ],
    )
    def deg_kernel(dstp_hbm, out_hbm, idx_v, zbuf, ones_v, acc_sh):
        c = lax.axis_index("c")
        s = lax.axis_index("s")
        wid = c * NTILE + s
        base = s * rpt

        def zfill(i, _):
            zbuf[pl.ds(i * 16, 16)] = jnp.zeros((16,), F32)
            return 0

        lax.fori_loop(0, rpt // 16, zfill, 0)

        def ofill(i, _):
            ones_v[pl.ds(i * 16, 16)] = jnp.ones((16,), F32)
            return 0

        lax.fori_loop(0, CH // 16, ofill, 0)

        pltpu.sync_copy(zbuf, acc_sh.at[pl.ds(base, rpt)])
        pltpu.sync_copy(dstp_hbm.at[wid], idx_v)
        plsc.subcore_barrier()

        def step(j, _):
            pltpu.sync_copy(ones_v, acc_sh.at[idx_v.at[j]], add=True)
            return 0

        lax.fori_loop(0, k_chunks, step, 0)
        plsc.subcore_barrier()
        pltpu.sync_copy(acc_sh.at[pl.ds(base, rpt)],
                        out_hbm.at[c, pl.ds(base, rpt)])

    return deg_kernel(dstp)


def _message_call(ypack, srcp, dstp, npad, k_chunks, femb):
    rpt = npad // NTILE
    fl = npad * femb
    flt = rpt * femb
    fp = femb // 2
    npk = npad * fp

    @functools.partial(
        pl.kernel,
        mesh=_sc_mesh(),
        out_type=jax.ShapeDtypeStruct((NSC, fl), F32),
        compiler_params=pltpu.CompilerParams(needs_layout_passes=False),
        scratch_types=[
            pltpu.VMEM((npk,), jnp.int32),
            pltpu.VMEM((k_chunks, CH), jnp.int32),
            pltpu.VMEM((k_chunks, CH), jnp.int32),
            pltpu.VMEM((4, femb, CH), jnp.int32),
            pltpu.VMEM((4, femb, CH), F32),
            pltpu.VMEM((2048,), F32),
            pltpu.VMEM_SHARED((fl,), F32),
            pltpu.SemaphoreType.DMA,
            pltpu.SemaphoreType.DMA,
            pltpu.SemaphoreType.DMA,
            pltpu.SemaphoreType.DMA,
            pltpu.SemaphoreType.DMA,
        ],
    )
    def msg_kernel(y_hbm, srcp_hbm, dstp_hbm, out_hbm,
                   ylo, sidx, didx, eidx_d, vals, zbuf, acc,
                   ysem, ssem0, ssem1, ssem2, ssem3):
        c = lax.axis_index("c")
        s = lax.axis_index("s")
        wid = c * NTILE + s
        fbase = s * flt

        ycp = pltpu.async_copy(y_hbm, ylo, ysem)

        def zfill(i, _):
            zbuf[pl.ds(i * 16, 16)] = jnp.zeros((16,), F32)
            return 0

        lax.fori_loop(0, 128, zfill, 0)
        for k in range(flt // 2048):
            pltpu.sync_copy(zbuf, acc.at[pl.ds(fbase + k * 2048, 2048)])
        pltpu.sync_copy(srcp_hbm.at[wid], sidx)
        pltpu.sync_copy(dstp_hbm.at[wid], didx)
        ycp.wait()
        plsc.subcore_barrier()

        ssems = (ssem0, ssem1, ssem2, ssem3)
        hmask = jnp.full((16,), -65536, dtype=jnp.int32)
        sh16 = jnp.full((16,), 16, dtype=jnp.int32)

        def build_vals(j, b):
            for g in range(CH // 16):
                sv = sidx[j, pl.ds(g * 16, 16)] * fp
                dv = didx[j, pl.ds(g * 16, 16)] * femb
                for l in range(femb):
                    eidx_d[b, l, pl.ds(g * 16, 16)] = dv + l
                for l in range(fp):
                    pv = plsc.load_gather(ylo, [sv + l])
                    flo = plsc.bitcast(lax.shift_left(pv, sh16), F32)
                    fhi = plsc.bitcast(lax.bitwise_and(pv, hmask), F32)
                    vals[b, 2 * l, pl.ds(g * 16, 16)] = flo
                    vals[b, 2 * l + 1, pl.ds(g * 16, 16)] = fhi

        def scatter(b):
            for l in range(femb):
                pltpu.async_copy(vals.at[b, l], acc.at[eidx_d.at[b, l]],
                                 ssems[b], add=True)

        def wait_scatter(b):
            for l in range(femb):
                pltpu.make_async_copy(vals.at[b, l],
                                      acc.at[eidx_d.at[b, l]],
                                      ssems[b]).wait()

        for b in (0, 1, 2, 3):
            build_vals(b, b)
            scatter(b)

        def step(i, _):
            for b in (0, 1, 2, 3):
                j = 4 * i + b

                @pl.when(j + 4 < k_chunks)
                def _():
                    wait_scatter(b)
                    build_vals(j + 4, b)
                    scatter(b)

            return 0

        lax.fori_loop(0, k_chunks // 4, step, 0)
        for b in (0, 1, 2, 3):
            wait_scatter(b)
        plsc.subcore_barrier()
        pltpu.sync_copy(acc.at[pl.ds(fbase, flt)],
                        out_hbm.at[c, pl.ds(fbase, flt)])

    return msg_kernel(ypack, srcp, dstp)


def _prep_call(x, W, degp, n, npad, femb):
    def body(x_ref, w_ref, degp_ref, y_ref, xw_ref, dis_ref):
        deg = degp_ref[0, :] + degp_ref[1, :] + 1.0
        dis = lax.rsqrt(deg)
        dis_ref[...] = dis
        xw = jnp.dot(x_ref[...], w_ref[...], preferred_element_type=F32)
        xw_ref[0:n, :] = xw
        xw_ref[n:npad, :] = jnp.zeros((npad - n, femb), F32)
        y_ref[0:n, :] = xw * dis[0:n, None]
        y_ref[n:npad, :] = jnp.zeros((npad - n, femb), F32)

    return pl.pallas_call(
        body,
        out_shape=(
            jax.ShapeDtypeStruct((npad, femb), F32),
            jax.ShapeDtypeStruct((npad, femb), F32),
            jax.ShapeDtypeStruct((npad,), F32),
        ),
    )(x, W, degp)


def _finish_call(sparts, dis, xw, b, n_real, npad, femb):
    def body(s_ref, dis_ref, xw_ref, b_ref, out_ref):
        dis = dis_ref[...][:, None]
        sv = s_ref[0] + s_ref[1]
        o = sv * dis + xw_ref[...] * dis * dis + b_ref[...][None, :]
        h = jnp.tanh(o)
        rows = lax.broadcasted_iota(jnp.int32, (npad, femb), 0)
        h = jnp.where(rows < n_real, h, 0.0)
        out_ref[...] = jnp.sum(h, axis=0, keepdims=True) * (1.0 / n_real)

    return pl.pallas_call(
        body,
        out_shape=jax.ShapeDtypeStruct((1, femb), F32),
    )(sparts, dis, xw, b)


def kernel(x, edge_index, W, b):
    n, dfeat = x.shape
    femb = W.shape[1]
    e = edge_index.shape[1]

    rpt = -(-n // NTILE)
    rpt = -(-rpt // CH) * CH
    npad = rpt * NTILE

    k_chunks = -(-e // (NW * CH))
    k_chunks += k_chunks % 2
    ep = NW * k_chunks * CH

    src = edge_index[0].astype(jnp.int32)
    dst = edge_index[1].astype(jnp.int32)
    padv = n + jnp.arange(ep - e, dtype=jnp.int32) % (npad - n)
    srcp = jnp.concatenate([src, padv]).reshape(NW, k_chunks, CH)
    dstp = jnp.concatenate([dst, padv]).reshape(NW, k_chunks, CH)

    degp = _degree_call(dstp, npad, k_chunks)
    y, xw, dis = _prep_call(x, W, degp, n, npad, femb)
    ypack = lax.bitcast_convert_type(
        y.astype(jnp.bfloat16).reshape(-1, 2), jnp.int32)
    sflat = _message_call(ypack, srcp, dstp, npad, k_chunks, femb)
    sparts = sflat.reshape(NSC, npad, femb)
    return _finish_call(sparts, dis, xw, b, n, npad, femb)


# speedup vs baseline: 1.1401x; 1.1401x over previous
import functools

import jax
import jax.numpy as jnp
from jax import lax
from jax.experimental import pallas as pl
from jax.experimental.pallas import tpu as pltpu
from jax.experimental.pallas import tpu_sc as plsc

NSC = 2
NTILE = 16
NW = NSC * NTILE
CH = 128
F32 = jnp.float32


def _sc_mesh():
    return plsc.VectorSubcoreMesh(core_axis_name="c", subcore_axis_name="s")


def _degree_call(dstp, npad, k_chunks):
    rpt = npad // NTILE

    @functools.partial(
        pl.kernel,
        mesh=_sc_mesh(),
        out_type=jax.ShapeDtypeStruct((NSC, npad), F32),
        scratch_types=[
            pltpu.VMEM((k_chunks, CH), jnp.int32),
            pltpu.VMEM((rpt,), F32),
            pltpu.VMEM((CH,), F32),
            pltpu.VMEM_SHARED((npad,), F32),
        ],
    )
    def deg_kernel(dstp_hbm, out_hbm, idx_v, zbuf, ones_v, acc_sh):
        c = lax.axis_index("c")
        s = lax.axis_index("s")
        wid = c * NTILE + s
        base = s * rpt

        def zfill(i, _):
            zbuf[pl.ds(i * 16, 16)] = jnp.zeros((16,), F32)
            return 0

        lax.fori_loop(0, rpt // 16, zfill, 0)

        def ofill(i, _):
            ones_v[pl.ds(i * 16, 16)] = jnp.ones((16,), F32)
            return 0

        lax.fori_loop(0, CH // 16, ofill, 0)

        pltpu.sync_copy(zbuf, acc_sh.at[pl.ds(base, rpt)])
        pltpu.sync_copy(dstp_hbm.at[wid], idx_v)
        plsc.subcore_barrier()

        def step(j, _):
            pltpu.sync_copy(ones_v, acc_sh.at[idx_v.at[j]], add=True)
            return 0

        lax.fori_loop(0, k_chunks, step, 0)
        plsc.subcore_barrier()
        pltpu.sync_copy(acc_sh.at[pl.ds(base, rpt)],
                        out_hbm.at[c, pl.ds(base, rpt)])

    return deg_kernel(dstp)


def _message_call(ypack, srcp, dstp, npad, k_chunks, femb):
    rpt = npad // NTILE
    fl = npad * femb
    flt = rpt * femb
    fp = femb // 2
    npk = npad * fp

    @functools.partial(
        pl.kernel,
        mesh=_sc_mesh(),
        out_type=jax.ShapeDtypeStruct((NSC, fl), F32),
        compiler_params=pltpu.CompilerParams(needs_layout_passes=False),
        scratch_types=[
            pltpu.VMEM((npk,), jnp.int32),
            pltpu.VMEM((k_chunks, CH), jnp.int32),
            pltpu.VMEM((k_chunks, CH), jnp.int32),
            pltpu.VMEM((2, femb, CH), jnp.int32),
            pltpu.VMEM((2, femb, CH), F32),
            pltpu.VMEM((2048,), F32),
            pltpu.VMEM_SHARED((fl,), F32),
            pltpu.SemaphoreType.DMA,
            pltpu.SemaphoreType.DMA,
            pltpu.SemaphoreType.DMA,
        ],
    )
    def msg_kernel(y_hbm, srcp_hbm, dstp_hbm, out_hbm,
                   ylo, sidx, didx, eidx_d, vals, zbuf, acc,
                   ysem, ssem0, ssem1):
        c = lax.axis_index("c")
        s = lax.axis_index("s")
        wid = c * NTILE + s
        fbase = s * flt

        ycp = pltpu.async_copy(y_hbm, ylo, ysem)

        def zfill(i, _):
            zbuf[pl.ds(i * 16, 16)] = jnp.zeros((16,), F32)
            return 0

        lax.fori_loop(0, 128, zfill, 0)
        for k in range(flt // 2048):
            pltpu.sync_copy(zbuf, acc.at[pl.ds(fbase + k * 2048, 2048)])
        pltpu.sync_copy(srcp_hbm.at[wid], sidx)
        pltpu.sync_copy(dstp_hbm.at[wid], didx)
        ycp.wait()
        plsc.subcore_barrier()

        ssems = (ssem0, ssem1)
        hmask = jnp.full((16,), -65536, dtype=jnp.int32)
        sh16 = jnp.full((16,), 16, dtype=jnp.int32)

        def build_vals(j, b):
            for g in range(CH // 16):
                sv = sidx[j, pl.ds(g * 16, 16)] * fp
                dv = didx[j, pl.ds(g * 16, 16)] * femb
                for l in range(femb):
                    eidx_d[b, l, pl.ds(g * 16, 16)] = dv + l
                for l in range(fp):
                    pv = plsc.load_gather(ylo, [sv + l])
                    flo = plsc.bitcast(lax.shift_left(pv, sh16), F32)
                    fhi = plsc.bitcast(lax.bitwise_and(pv, hmask), F32)
                    vals[b, 2 * l, pl.ds(g * 16, 16)] = flo
                    vals[b, 2 * l + 1, pl.ds(g * 16, 16)] = fhi

        def scatter(b):
            for l in range(femb):
                pltpu.async_copy(vals.at[b, l], acc.at[eidx_d.at[b, l]],
                                 ssems[b], add=True)

        def wait_scatter(b):
            for l in range(femb):
                pltpu.make_async_copy(vals.at[b, l],
                                      acc.at[eidx_d.at[b, l]],
                                      ssems[b]).wait()

        for b in (0, 1):
            build_vals(b, b)
            scatter(b)

        def step(i, _):
            for b in (0, 1):
                j = 2 * i + b

                @pl.when(j + 2 < k_chunks)
                def _():
                    wait_scatter(b)
                    build_vals(j + 2, b)
                    scatter(b)

            return 0

        lax.fori_loop(0, k_chunks // 2, step, 0)
        for b in (0, 1):
            wait_scatter(b)
        plsc.subcore_barrier()
        pltpu.sync_copy(acc.at[pl.ds(fbase, flt)],
                        out_hbm.at[c, pl.ds(fbase, flt)])

    return msg_kernel(ypack, srcp, dstp)


def _prep_call(x, W, degp, n, npad, femb):
    def body(x_ref, w_ref, degp_ref, y_ref, xw_ref, dis_ref):
        deg = degp_ref[0, :] + degp_ref[1, :] + 1.0
        dis = lax.rsqrt(deg)
        dis_ref[...] = dis
        xw = jnp.dot(x_ref[...], w_ref[...], preferred_element_type=F32)
        xw_ref[0:n, :] = xw
        xw_ref[n:npad, :] = jnp.zeros((npad - n, femb), F32)
        y_ref[0:n, :] = xw * dis[0:n, None]
        y_ref[n:npad, :] = jnp.zeros((npad - n, femb), F32)

    return pl.pallas_call(
        body,
        out_shape=(
            jax.ShapeDtypeStruct((npad, femb), F32),
            jax.ShapeDtypeStruct((npad, femb), F32),
            jax.ShapeDtypeStruct((npad,), F32),
        ),
    )(x, W, degp)


def _finish_call(sparts, dis, xw, b, n_real, npad, femb):
    def body(s_ref, dis_ref, xw_ref, b_ref, out_ref):
        dis = dis_ref[...][:, None]
        sv = s_ref[0] + s_ref[1]
        o = sv * dis + xw_ref[...] * dis * dis + b_ref[...][None, :]
        h = jnp.tanh(o)
        rows = lax.broadcasted_iota(jnp.int32, (npad, femb), 0)
        h = jnp.where(rows < n_real, h, 0.0)
        out_ref[...] = jnp.sum(h, axis=0, keepdims=True) * (1.0 / n_real)

    return pl.pallas_call(
        body,
        out_shape=jax.ShapeDtypeStruct((1, femb), F32),
    )(sparts, dis, xw, b)


def kernel(x, edge_index, W, b):
    n, dfeat = x.shape
    femb = W.shape[1]
    e = edge_index.shape[1]

    rpt = -(-n // NTILE)
    rpt = -(-rpt // CH) * CH
    npad = rpt * NTILE

    k_chunks = -(-e // (NW * CH))
    k_chunks += k_chunks % 2
    ep = NW * k_chunks * CH

    src = edge_index[0].astype(jnp.int32)
    dst = edge_index[1].astype(jnp.int32)
    padv = n + jnp.arange(ep - e, dtype=jnp.int32) % (npad - n)
    srcp = jnp.concatenate([src, padv]).reshape(NW, k_chunks, CH)
    dstp = jnp.concatenate([dst, padv]).reshape(NW, k_chunks, CH)

    degp = _degree_call(dstp, npad, k_chunks)
    y, xw, dis = _prep_call(x, W, degp, n, npad, femb)
    ypack = lax.bitcast_convert_type(
        y.astype(jnp.bfloat16).reshape(-1, 2), jnp.int32)
    sflat = _message_call(ypack, srcp, dstp, npad, k_chunks, femb)
    sparts = sflat.reshape(NSC, npad, femb)
    return _finish_call(sparts, dis, xw, b, n, npad, femb)
